# Initial kernel scaffold; baseline (speedup 1.0000x reference)
import jax
import jax.numpy as jnp
from jax.experimental import pallas as pl


def kernel(x, edge_index, w_src0, w_dst0, a_src0, a_dst0, b0, wl0, bl0, w1, a_src1, a_dst1, b1, wl1, bl1, ln_w, ln_b, wp, bp):
    raise NotImplementedError("write your pallas kernel here")


# speedup vs baseline: 22.6307x; 22.6307x over previous
import functools

import jax
import jax.numpy as jnp
from jax import lax
from jax.experimental import pallas as pl
from jax.experimental.pallas import tpu as pltpu
from jax.experimental.pallas import tpu_sc as plsc

_NC = 2
_NS = 16
_NW = _NC * _NS
_L = 16

_SS = 80
_JROWS = 1
_CH = _SS * _JROWS


def _splat(vec, j):
  idx = jnp.full((_L, 1), j, jnp.int32)
  dn = lax.GatherDimensionNumbers(
      offset_dims=(), collapsed_slice_dims=(0,), start_index_map=(0,))
  return lax.gather(vec, idx, dn, (1,),
                    mode=lax.GatherScatterMode.PROMISE_IN_BOUNDS)


def _make_sc_edge_kernel(n_nodes, n_edges):
  N = n_nodes
  EP = n_edges // _NW
  assert EP % _CH == 0
  n_chunks = EP // _CH
  RPT = (N // _NS) & ~7
  TAIL = N - _NS * RPT
  ZC = 2000

  mesh = plsc.VectorSubcoreMesh(core_axis_name="c", subcore_axis_name="s",
                                num_cores=_NC, num_subcores=_NS)

  @functools.partial(
      pl.kernel,
      out_type=(
          jax.ShapeDtypeStruct((_NC, N, 128), jnp.float32),
          jax.ShapeDtypeStruct((_NC, N), jnp.float32),
      ),
      mesh=mesh,
      scratch_types=dict(
          als_v=pltpu.VMEM((N,), jnp.float32),
          ald_v=pltpu.VMEM((N,), jnp.float32),
          src_v=tuple(pltpu.VMEM((_SS,), jnp.int32) for _ in range(_JROWS)),
          dst_v=tuple(pltpu.VMEM((_SS,), jnp.int32) for _ in range(_JROWS)),
          ex_v=tuple(pltpu.VMEM((_SS,), jnp.float32) for _ in range(_JROWS)),
          rows_v=pltpu.VMEM((_CH, 128), jnp.float32),
          den_stage=pltpu.VMEM((ZC,), jnp.float32),
          out_sp=pltpu.VMEM_SHARED((N, 128), jnp.float32),
          den_sp=pltpu.VMEM_SHARED((N,), jnp.float32),
          sem=pltpu.SemaphoreType.DMA,
      ),
      compiler_params=pltpu.CompilerParams(use_tc_tiling_on_sc=False,
                                           needs_layout_passes=False),
  )
  def sc_kernel(hs_hbm, als_hbm, ald_hbm, src_hbm, dst_hbm,
                out_hbm, den_hbm, *, als_v, ald_v, src_v, dst_v, ex_v,
                rows_v, den_stage, out_sp, den_sp, sem):
    cid = lax.axis_index("c")
    sid = lax.axis_index("s")
    wid = sid * _NC + cid

    zero16 = jnp.zeros((_L,), jnp.float32)

    pltpu.sync_copy(als_hbm, als_v)
    pltpu.sync_copy(ald_hbm, ald_v)

    def zrow(i, _):
      for r in range(8):
        rows_v[i, pl.ds(r * _L, _L)] = zero16
      return 0
    lax.fori_loop(0, _CH, zrow, 0)
    r0 = sid * RPT
    nfull = RPT // _CH
    rrem = RPT % _CH

    def zcp(k, _):
      pltpu.sync_copy(rows_v, out_sp.at[pl.ds(r0 + k * _CH, _CH)])
      return 0
    lax.fori_loop(0, nfull, zcp, 0)
    if rrem:
      pltpu.sync_copy(rows_v.at[pl.ds(0, rrem)],
                      out_sp.at[pl.ds(r0 + nfull * _CH, rrem)])

    @pl.when(sid == _NS - 1)
    def _():
      pltpu.sync_copy(rows_v.at[pl.ds(0, TAIL)],
                      out_sp.at[pl.ds(N - TAIL, TAIL)])

    @pl.when(sid < N // ZC)
    def _():
      def zden(i, _):
        den_stage[pl.ds(i * _L, _L)] = zero16
        return 0
      lax.fori_loop(0, ZC // _L, zden, 0)
      pltpu.sync_copy(den_stage, den_sp.at[pl.ds(sid * ZC, ZC)])

    plsc.subcore_barrier()

    idx_row0 = wid * (EP // _SS)

    def chunk_body(k, _):
      row0 = idx_row0 + k * _JROWS
      for j in range(_JROWS):
        pltpu.sync_copy(src_hbm.at[row0 + j], src_v[j])
        pltpu.sync_copy(dst_hbm.at[row0 + j], dst_v[j])

      handles = [
          pltpu.async_copy(hs_hbm.at[src_v[j]],
                           rows_v.at[pl.ds(j * _SS, _SS)], sem)
          for j in range(_JROWS)
      ]
      for h in handles:
        h.wait()

      for j in range(_JROWS):
        def grp(g, _, j=j):
          s16 = src_v[j][pl.ds(g * _L, _L)]
          d16 = dst_v[j][pl.ds(g * _L, _L)]
          als16 = plsc.load_gather(als_v, [s16])
          ald16 = plsc.load_gather(ald_v, [d16])
          e16 = als16 + ald16
          e16 = jnp.where(e16 >= 0.0, e16, 0.2 * e16)
          ex16 = jnp.exp(e16)
          ex_v[j][pl.ds(g * _L, _L)] = ex16
          for jj in range(_L):
            e0 = j * _SS + g * _L + jj
            sp = _splat(ex16, jj)
            for r in range(8):
              sl = pl.ds(r * _L, _L)
              rows_v[e0, sl] = rows_v[e0, sl] * sp
          return 0
        lax.fori_loop(0, _SS // _L, grp, 0)

      for j in range(_JROWS):
        pltpu.sync_copy(rows_v.at[pl.ds(j * _SS, _SS)],
                        out_sp.at[dst_v[j]], add=True)
        pltpu.sync_copy(ex_v[j], den_sp.at[dst_v[j]], add=True)
      return 0

    lax.fori_loop(0, n_chunks, chunk_body, 0)

    plsc.subcore_barrier()

    def ocp(k, _):
      pltpu.sync_copy(out_sp.at[pl.ds(r0 + k * _CH, _CH)], rows_v)
      pltpu.sync_copy(rows_v, out_hbm.at[cid, pl.ds(r0 + k * _CH, _CH)])
      return 0
    lax.fori_loop(0, nfull, ocp, 0)
    if rrem:
      pltpu.sync_copy(out_sp.at[pl.ds(r0 + nfull * _CH, rrem)],
                      rows_v.at[pl.ds(0, rrem)])
      pltpu.sync_copy(rows_v.at[pl.ds(0, rrem)],
                      out_hbm.at[cid, pl.ds(r0 + nfull * _CH, rrem)])

    @pl.when(sid == _NS - 1)
    def _():
      pltpu.sync_copy(out_sp.at[pl.ds(N - TAIL, TAIL)],
                      rows_v.at[pl.ds(0, TAIL)])
      pltpu.sync_copy(rows_v.at[pl.ds(0, TAIL)],
                      out_hbm.at[cid, pl.ds(N - TAIL, TAIL)])

    @pl.when(sid < N // ZC)
    def _():
      pltpu.sync_copy(den_sp.at[pl.ds(sid * ZC, ZC)], den_stage)
      pltpu.sync_copy(den_stage, den_hbm.at[cid, pl.ds(sid * ZC, ZC)])

  return sc_kernel


_B = 1000


def _tc1_body(x, wsrc, wdst, a_s, a_d, hs_o, als_o, ald_o):
  xb = x[...]
  hs = jnp.dot(xb, wsrc[...], preferred_element_type=jnp.float32)
  hs_o[...] = hs
  als_o[...] = jnp.dot(hs, a_s[...], preferred_element_type=jnp.float32)
  vd = jnp.dot(wdst[...], a_d[...], preferred_element_type=jnp.float32)
  ald_o[...] = jnp.dot(xb, vd, preferred_element_type=jnp.float32)


def _tc2_body(outA, outB, denA, denB, x, b0, wl0, bl0, w1, as1, ad1,
              h_o, hs1_o, als1_o, ald1_o):
  d = denA[...] + denB[...] + 1e-16
  gat = (outA[...] + outB[...]) / d
  xb = x[...]
  h = jnp.maximum(
      gat + b0[...]
      + jnp.dot(xb, wl0[...], preferred_element_type=jnp.float32)
      + bl0[...], 0.0)
  h_o[...] = h
  hs1 = jnp.dot(h, w1[...], preferred_element_type=jnp.float32)
  hs1_o[...] = hs1
  als1_o[...] = jnp.dot(hs1, as1[...], preferred_element_type=jnp.float32)
  vd1 = jnp.dot(w1[...], ad1[...], preferred_element_type=jnp.float32)
  ald1_o[...] = jnp.dot(h, vd1, preferred_element_type=jnp.float32)


def _tc3_body(outA, outB, denA, denB, h, b1, wl1, bl1,
              h2_o, ssum_o, sqsum_o):
  d = denA[...] + denB[...] + 1e-16
  gat = (outA[...] + outB[...]) / d
  h2 = (gat + b1[...]
        + jnp.dot(h[...], wl1[...], preferred_element_type=jnp.float32)
        + bl1[...])
  h2_o[...] = h2
  i = pl.program_id(0)
  ssum_o[pl.ds(i, 1), :] = jnp.sum(h2, axis=0, keepdims=True)
  sqsum_o[pl.ds(i, 1), :] = jnp.sum(h2 * h2, axis=0, keepdims=True)


def _tc4_body(nd, h2, ssum, sqsum, ln_w, ln_b, wp, bp, y_o):
  mu = jnp.sum(ssum[...]) / nd
  var = jnp.sum(sqsum[...]) / nd - mu * mu
  inv = lax.rsqrt(var + 1e-5)
  hn = (h2[...] - mu) * inv * ln_w[...] + ln_b[...]
  y_o[...] = jnp.dot(hn, wp[...], preferred_element_type=jnp.float32) + bp[...]


def kernel(x, edge_index, w_src0, w_dst0, a_src0, a_dst0, b0, wl0, bl0,
           w1, a_src1, a_dst1, b1, wl1, bl1, ln_w, ln_b, wp, bp):
  N, D = x.shape
  E = edge_index.shape[1]
  nb = N // _B

  src2 = edge_index[0].reshape(E // _SS, _SS)
  dst2 = edge_index[1].reshape(E // _SS, _SS)

  a_s0c = a_src0.reshape(D, 1)
  a_d0c = a_dst0.reshape(D, 1)
  a_s1c = a_src1.reshape(D, 1)
  a_d1c = a_dst1.reshape(D, 1)
  b0r = b0.reshape(1, D)
  bl0r = bl0.reshape(1, D)
  b1r = b1.reshape(1, D)
  bl1r = bl1.reshape(1, D)
  ln_wr = ln_w.reshape(1, D)
  ln_br = ln_b.reshape(1, D)
  bpr = bp.reshape(1, 1)

  row_spec = pl.BlockSpec((_B, D), lambda i: (i, 0))
  col1_spec = pl.BlockSpec((_B, 1), lambda i: (i, 0))
  w_spec = pl.BlockSpec((D, D), lambda i: (0, 0))
  v_spec = pl.BlockSpec((D, 1), lambda i: (0, 0))
  r_spec = pl.BlockSpec((1, D), lambda i: (0, 0))

  hs0, als0, ald0 = pl.pallas_call(
      _tc1_body,
      grid=(nb,),
      in_specs=[row_spec, w_spec, w_spec, v_spec, v_spec],
      out_specs=[row_spec, col1_spec, col1_spec],
      out_shape=[
          jax.ShapeDtypeStruct((N, D), jnp.float32),
          jax.ShapeDtypeStruct((N, 1), jnp.float32),
          jax.ShapeDtypeStruct((N, 1), jnp.float32),
      ],
  )(x, w_src0, w_dst0, a_s0c, a_d0c)

  sc_edge = _make_sc_edge_kernel(N, E)

  out2_0, den2_0 = sc_edge(hs0, als0.reshape(N), ald0.reshape(N),
                           src2, dst2)

  h, hs1, als1, ald1 = pl.pallas_call(
      _tc2_body,
      grid=(nb,),
      in_specs=[row_spec, row_spec, col1_spec, col1_spec, row_spec,
                r_spec, w_spec, r_spec, w_spec, v_spec, v_spec],
      out_specs=[row_spec, row_spec, col1_spec, col1_spec],
      out_shape=[
          jax.ShapeDtypeStruct((N, D), jnp.float32),
          jax.ShapeDtypeStruct((N, D), jnp.float32),
          jax.ShapeDtypeStruct((N, 1), jnp.float32),
          jax.ShapeDtypeStruct((N, 1), jnp.float32),
      ],
  )(out2_0[0], out2_0[1], den2_0[0].reshape(N, 1), den2_0[1].reshape(N, 1),
    x, b0r, wl0, bl0r, w1, a_s1c, a_d1c)

  out2_1, den2_1 = sc_edge(hs1, als1.reshape(N), ald1.reshape(N),
                           src2, dst2)

  h2, ssum, sqsum = pl.pallas_call(
      _tc3_body,
      grid=(nb,),
      in_specs=[row_spec, row_spec, col1_spec, col1_spec, row_spec,
                r_spec, w_spec, r_spec],
      out_specs=[row_spec, pl.BlockSpec((nb, D), lambda i: (0, 0)),
                 pl.BlockSpec((nb, D), lambda i: (0, 0))],
      out_shape=[
          jax.ShapeDtypeStruct((N, D), jnp.float32),
          jax.ShapeDtypeStruct((nb, D), jnp.float32),
          jax.ShapeDtypeStruct((nb, D), jnp.float32),
      ],
  )(out2_1[0], out2_1[1], den2_1[0].reshape(N, 1), den2_1[1].reshape(N, 1),
    h, b1r, wl1, bl1r)

  y = pl.pallas_call(
      functools.partial(_tc4_body, float(N * D)),
      grid=(nb,),
      in_specs=[row_spec, pl.BlockSpec((nb, D), lambda i: (0, 0)),
                pl.BlockSpec((nb, D), lambda i: (0, 0)),
                r_spec, r_spec, v_spec, pl.BlockSpec((1, 1), lambda i: (0, 0))],
      out_specs=[col1_spec],
      out_shape=[jax.ShapeDtypeStruct((N, 1), jnp.float32)],
  )(h2, ssum, sqsum, ln_wr, ln_br, wp, bpr)[0]

  return y


# speedup vs baseline: 31.6177x; 1.3971x over previous
import functools

import jax
import jax.numpy as jnp
from jax import lax
from jax.experimental import pallas as pl
from jax.experimental.pallas import tpu as pltpu
from jax.experimental.pallas import tpu_sc as plsc

_NC = 2
_NS = 16
_NW = _NC * _NS
_L = 16

_SS = 80
_JROWS = 1
_CH = _SS * _JROWS


def _splat(vec, j):
  idx = jnp.full((_L, 1), j, jnp.int32)
  dn = lax.GatherDimensionNumbers(
      offset_dims=(), collapsed_slice_dims=(0,), start_index_map=(0,))
  return lax.gather(vec, idx, dn, (1,),
                    mode=lax.GatherScatterMode.PROMISE_IN_BOUNDS)


def _make_sc_edge_kernel(n_nodes, n_edges):
  N = n_nodes
  EP = n_edges // _NW
  assert EP % _CH == 0
  n_chunks = EP // _CH
  RPT = (N // _NS) & ~7
  TAIL = N - _NS * RPT
  ZC = 2000

  mesh = plsc.VectorSubcoreMesh(core_axis_name="c", subcore_axis_name="s",
                                num_cores=_NC, num_subcores=_NS)

  @functools.partial(
      pl.kernel,
      out_type=(
          jax.ShapeDtypeStruct((_NC, N, 128), jnp.float32),
          jax.ShapeDtypeStruct((_NC, N), jnp.float32),
      ),
      mesh=mesh,
      scratch_types=dict(
          als_v=pltpu.VMEM((N,), jnp.float32),
          ald_v=pltpu.VMEM((N,), jnp.float32),
          src_v=tuple(pltpu.VMEM((_SS,), jnp.int32) for _ in range(2)),
          dst_v=tuple(pltpu.VMEM((_SS,), jnp.int32) for _ in range(2)),
          ex_v=tuple(pltpu.VMEM((_SS,), jnp.float32) for _ in range(2)),
          rows_v=tuple(pltpu.VMEM((_CH, 128), jnp.float32) for _ in range(2)),
          den_stage=pltpu.VMEM((ZC,), jnp.float32),
          out_sp=pltpu.VMEM_SHARED((N, 128), jnp.float32),
          den_sp=pltpu.VMEM_SHARED((N,), jnp.float32),
          sem=tuple(pltpu.SemaphoreType.DMA for _ in range(2)),
      ),
      compiler_params=pltpu.CompilerParams(use_tc_tiling_on_sc=False,
                                           needs_layout_passes=False),
  )
  def sc_kernel(hs_hbm, als_hbm, ald_hbm, src_hbm, dst_hbm,
                out_hbm, den_hbm, *, als_v, ald_v, src_v, dst_v, ex_v,
                rows_v, den_stage, out_sp, den_sp, sem):
    cid = lax.axis_index("c")
    sid = lax.axis_index("s")
    wid = sid * _NC + cid

    zero16 = jnp.zeros((_L,), jnp.float32)

    pltpu.sync_copy(als_hbm, als_v)
    pltpu.sync_copy(ald_hbm, ald_v)

    def zrow(i, _):
      for r in range(8):
        rows_v[0][i, pl.ds(r * _L, _L)] = zero16
      return 0
    lax.fori_loop(0, _CH, zrow, 0)
    r0 = sid * RPT
    nfull = RPT // _CH
    rrem = RPT % _CH

    def zcp(k, _):
      pltpu.sync_copy(rows_v[0], out_sp.at[pl.ds(r0 + k * _CH, _CH)])
      return 0
    lax.fori_loop(0, nfull, zcp, 0)
    if rrem:
      pltpu.sync_copy(rows_v[0].at[pl.ds(0, rrem)],
                      out_sp.at[pl.ds(r0 + nfull * _CH, rrem)])

    @pl.when(sid == _NS - 1)
    def _():
      pltpu.sync_copy(rows_v[0].at[pl.ds(0, TAIL)],
                      out_sp.at[pl.ds(N - TAIL, TAIL)])

    @pl.when(sid < N // ZC)
    def _():
      def zden(i, _):
        den_stage[pl.ds(i * _L, _L)] = zero16
        return 0
      lax.fori_loop(0, ZC // _L, zden, 0)
      pltpu.sync_copy(den_stage, den_sp.at[pl.ds(sid * ZC, ZC)])

    plsc.subcore_barrier()

    idx_row0 = wid * (EP // _SS)

    def load_and_fire(b, c):
      pltpu.sync_copy(src_hbm.at[idx_row0 + c], src_v[b])
      pltpu.sync_copy(dst_hbm.at[idx_row0 + c], dst_v[b])
      pltpu.async_copy(hs_hbm.at[src_v[b]], rows_v[b], sem[b])

    def wait_gather(b):
      pltpu.make_async_copy(hs_hbm.at[src_v[b]], rows_v[b], sem[b]).wait()

    def compute_and_scatter(b):
      def grp(g, _):
        s16 = src_v[b][pl.ds(g * _L, _L)]
        d16 = dst_v[b][pl.ds(g * _L, _L)]
        als16 = plsc.load_gather(als_v, [s16])
        ald16 = plsc.load_gather(ald_v, [d16])
        e16 = als16 + ald16
        e16 = jnp.where(e16 >= 0.0, e16, 0.2 * e16)
        ex16 = jnp.exp(e16)
        ex_v[b][pl.ds(g * _L, _L)] = ex16
        for jj in range(_L):
          e0 = g * _L + jj
          sp = _splat(ex16, jj)
          for r in range(8):
            sl = pl.ds(r * _L, _L)
            rows_v[b][e0, sl] = rows_v[b][e0, sl] * sp
        return 0
      lax.fori_loop(0, _SS // _L, grp, 0)
      pltpu.sync_copy(rows_v[b], out_sp.at[dst_v[b]], add=True)
      pltpu.sync_copy(ex_v[b], den_sp.at[dst_v[b]], add=True)

    load_and_fire(0, 0)

    def pair_body(i, _):
      c0 = 2 * i

      @pl.when(c0 + 1 < n_chunks)
      def _():
        load_and_fire(1, c0 + 1)
      wait_gather(0)
      compute_and_scatter(0)

      @pl.when(c0 + 1 < n_chunks)
      def _():
        @pl.when(c0 + 2 < n_chunks)
        def _():
          load_and_fire(0, c0 + 2)
        wait_gather(1)
        compute_and_scatter(1)
      return 0

    lax.fori_loop(0, (n_chunks + 1) // 2, pair_body, 0)

    plsc.subcore_barrier()

    def ocp(k, _):
      pltpu.sync_copy(out_sp.at[pl.ds(r0 + k * _CH, _CH)], rows_v[0])
      pltpu.sync_copy(rows_v[0], out_hbm.at[cid, pl.ds(r0 + k * _CH, _CH)])
      return 0
    lax.fori_loop(0, nfull, ocp, 0)
    if rrem:
      pltpu.sync_copy(out_sp.at[pl.ds(r0 + nfull * _CH, rrem)],
                      rows_v[0].at[pl.ds(0, rrem)])
      pltpu.sync_copy(rows_v[0].at[pl.ds(0, rrem)],
                      out_hbm.at[cid, pl.ds(r0 + nfull * _CH, rrem)])

    @pl.when(sid == _NS - 1)
    def _():
      pltpu.sync_copy(out_sp.at[pl.ds(N - TAIL, TAIL)],
                      rows_v[1].at[pl.ds(0, TAIL)])
      pltpu.sync_copy(rows_v[1].at[pl.ds(0, TAIL)],
                      out_hbm.at[cid, pl.ds(N - TAIL, TAIL)])

    @pl.when(sid < N // ZC)
    def _():
      pltpu.sync_copy(den_sp.at[pl.ds(sid * ZC, ZC)], den_stage)
      pltpu.sync_copy(den_stage, den_hbm.at[cid, pl.ds(sid * ZC, ZC)])

  return sc_kernel


_B = 1000


def _tc1_body(x, wsrc, wdst, a_s, a_d, hs_o, als_o, ald_o):
  xb = x[...]
  hs = jnp.dot(xb, wsrc[...], preferred_element_type=jnp.float32)
  hs_o[...] = hs
  als_o[...] = jnp.dot(hs, a_s[...], preferred_element_type=jnp.float32)
  vd = jnp.dot(wdst[...], a_d[...], preferred_element_type=jnp.float32)
  ald_o[...] = jnp.dot(xb, vd, preferred_element_type=jnp.float32)


def _tc2_body(outA, outB, denA, denB, x, b0, wl0, bl0, w1, as1, ad1,
              h_o, hs1_o, als1_o, ald1_o):
  d = denA[...] + denB[...] + 1e-16
  gat = (outA[...] + outB[...]) / d
  xb = x[...]
  h = jnp.maximum(
      gat + b0[...]
      + jnp.dot(xb, wl0[...], preferred_element_type=jnp.float32)
      + bl0[...], 0.0)
  h_o[...] = h
  hs1 = jnp.dot(h, w1[...], preferred_element_type=jnp.float32)
  hs1_o[...] = hs1
  als1_o[...] = jnp.dot(hs1, as1[...], preferred_element_type=jnp.float32)
  vd1 = jnp.dot(w1[...], ad1[...], preferred_element_type=jnp.float32)
  ald1_o[...] = jnp.dot(h, vd1, preferred_element_type=jnp.float32)


def _tc3_body(outA, outB, denA, denB, h, b1, wl1, bl1,
              h2_o, ssum_o, sqsum_o):
  d = denA[...] + denB[...] + 1e-16
  gat = (outA[...] + outB[...]) / d
  h2 = (gat + b1[...]
        + jnp.dot(h[...], wl1[...], preferred_element_type=jnp.float32)
        + bl1[...])
  h2_o[...] = h2
  i = pl.program_id(0)
  ssum_o[pl.ds(i, 1), :] = jnp.sum(h2, axis=0, keepdims=True)
  sqsum_o[pl.ds(i, 1), :] = jnp.sum(h2 * h2, axis=0, keepdims=True)


def _tc4_body(nd, h2, ssum, sqsum, ln_w, ln_b, wp, bp, y_o):
  mu = jnp.sum(ssum[...]) / nd
  var = jnp.sum(sqsum[...]) / nd - mu * mu
  inv = lax.rsqrt(var + 1e-5)
  hn = (h2[...] - mu) * inv * ln_w[...] + ln_b[...]
  y_o[...] = jnp.dot(hn, wp[...], preferred_element_type=jnp.float32) + bp[...]


def kernel(x, edge_index, w_src0, w_dst0, a_src0, a_dst0, b0, wl0, bl0,
           w1, a_src1, a_dst1, b1, wl1, bl1, ln_w, ln_b, wp, bp):
  N, D = x.shape
  E = edge_index.shape[1]
  nb = N // _B

  src2 = edge_index[0].reshape(E // _SS, _SS)
  dst2 = edge_index[1].reshape(E // _SS, _SS)

  a_s0c = a_src0.reshape(D, 1)
  a_d0c = a_dst0.reshape(D, 1)
  a_s1c = a_src1.reshape(D, 1)
  a_d1c = a_dst1.reshape(D, 1)
  b0r = b0.reshape(1, D)
  bl0r = bl0.reshape(1, D)
  b1r = b1.reshape(1, D)
  bl1r = bl1.reshape(1, D)
  ln_wr = ln_w.reshape(1, D)
  ln_br = ln_b.reshape(1, D)
  bpr = bp.reshape(1, 1)

  row_spec = pl.BlockSpec((_B, D), lambda i: (i, 0))
  col1_spec = pl.BlockSpec((_B, 1), lambda i: (i, 0))
  w_spec = pl.BlockSpec((D, D), lambda i: (0, 0))
  v_spec = pl.BlockSpec((D, 1), lambda i: (0, 0))
  r_spec = pl.BlockSpec((1, D), lambda i: (0, 0))

  hs0, als0, ald0 = pl.pallas_call(
      _tc1_body,
      grid=(nb,),
      in_specs=[row_spec, w_spec, w_spec, v_spec, v_spec],
      out_specs=[row_spec, col1_spec, col1_spec],
      out_shape=[
          jax.ShapeDtypeStruct((N, D), jnp.float32),
          jax.ShapeDtypeStruct((N, 1), jnp.float32),
          jax.ShapeDtypeStruct((N, 1), jnp.float32),
      ],
  )(x, w_src0, w_dst0, a_s0c, a_d0c)

  sc_edge = _make_sc_edge_kernel(N, E)

  out2_0, den2_0 = sc_edge(hs0, als0.reshape(N), ald0.reshape(N),
                           src2, dst2)

  h, hs1, als1, ald1 = pl.pallas_call(
      _tc2_body,
      grid=(nb,),
      in_specs=[row_spec, row_spec, col1_spec, col1_spec, row_spec,
                r_spec, w_spec, r_spec, w_spec, v_spec, v_spec],
      out_specs=[row_spec, row_spec, col1_spec, col1_spec],
      out_shape=[
          jax.ShapeDtypeStruct((N, D), jnp.float32),
          jax.ShapeDtypeStruct((N, D), jnp.float32),
          jax.ShapeDtypeStruct((N, 1), jnp.float32),
          jax.ShapeDtypeStruct((N, 1), jnp.float32),
      ],
  )(out2_0[0], out2_0[1], den2_0[0].reshape(N, 1), den2_0[1].reshape(N, 1),
    x, b0r, wl0, bl0r, w1, a_s1c, a_d1c)

  out2_1, den2_1 = sc_edge(hs1, als1.reshape(N), ald1.reshape(N),
                           src2, dst2)

  h2, ssum, sqsum = pl.pallas_call(
      _tc3_body,
      grid=(nb,),
      in_specs=[row_spec, row_spec, col1_spec, col1_spec, row_spec,
                r_spec, w_spec, r_spec],
      out_specs=[row_spec, pl.BlockSpec((nb, D), lambda i: (0, 0)),
                 pl.BlockSpec((nb, D), lambda i: (0, 0))],
      out_shape=[
          jax.ShapeDtypeStruct((N, D), jnp.float32),
          jax.ShapeDtypeStruct((nb, D), jnp.float32),
          jax.ShapeDtypeStruct((nb, D), jnp.float32),
      ],
  )(out2_1[0], out2_1[1], den2_1[0].reshape(N, 1), den2_1[1].reshape(N, 1),
    h, b1r, wl1, bl1r)

  y = pl.pallas_call(
      functools.partial(_tc4_body, float(N * D)),
      grid=(nb,),
      in_specs=[row_spec, pl.BlockSpec((nb, D), lambda i: (0, 0)),
                pl.BlockSpec((nb, D), lambda i: (0, 0)),
                r_spec, r_spec, v_spec, pl.BlockSpec((1, 1), lambda i: (0, 0))],
      out_specs=[col1_spec],
      out_shape=[jax.ShapeDtypeStruct((N, 1), jnp.float32)],
  )(h2, ssum, sqsum, ln_wr, ln_br, wp, bpr)[0]

  return y


# speedup vs baseline: 37.7015x; 1.1924x over previous
import functools

import jax
import jax.numpy as jnp
from jax import lax
from jax.experimental import pallas as pl
from jax.experimental.pallas import tpu as pltpu
from jax.experimental.pallas import tpu_sc as plsc

_NC = 2
_NS = 16
_NW = _NC * _NS
_L = 16

_SS = 80
_JROWS = 1
_CH = _SS * _JROWS


_SB = 25


def _make_sc_edge_kernel(n_nodes, n_edges):
  N = n_nodes
  EP = n_edges // _NW
  assert EP % _CH == 0
  n_chunks = EP // _CH
  RPT = (N // _NS) & ~7
  TAIL = N - _NS * RPT
  ZC = 2000

  mesh = plsc.VectorSubcoreMesh(core_axis_name="c", subcore_axis_name="s",
                                num_cores=_NC, num_subcores=_NS)

  @functools.partial(
      pl.kernel,
      out_type=(
          jax.ShapeDtypeStruct((_NC, N, 128), jnp.float32),
          jax.ShapeDtypeStruct((_NC, N), jnp.float32),
      ),
      mesh=mesh,
      scratch_types=dict(
          src_big=pltpu.VMEM((_SB, _SS), jnp.int32),
          dst_big=pltpu.VMEM((_SB, _SS), jnp.int32),
          als_g=tuple(pltpu.VMEM((_SS,), jnp.float32) for _ in range(2)),
          ald_g=tuple(pltpu.VMEM((_SS,), jnp.float32) for _ in range(2)),
          ex_v=tuple(pltpu.VMEM((_SS,), jnp.float32) for _ in range(2)),
          rows_v=tuple(pltpu.VMEM((_CH, 128), jnp.float32) for _ in range(2)),
          den_stage=pltpu.VMEM((ZC,), jnp.float32),
          out_sp=pltpu.VMEM_SHARED((N, 128), jnp.float32),
          den_sp=pltpu.VMEM_SHARED((N,), jnp.float32),
          sem=tuple(pltpu.SemaphoreType.DMA for _ in range(2)),
      ),
      compiler_params=pltpu.CompilerParams(use_tc_tiling_on_sc=False,
                                           needs_layout_passes=False),
  )
  def sc_kernel(hs_hbm, als_hbm, ald_hbm, src_hbm, dst_hbm,
                out_hbm, den_hbm, *, src_big, dst_big, als_g, ald_g, ex_v,
                rows_v, den_stage, out_sp, den_sp, sem):
    cid = lax.axis_index("c")
    sid = lax.axis_index("s")
    wid = sid * _NC + cid

    zero16 = jnp.zeros((_L,), jnp.float32)

    def zrow(i, _):
      for r in range(8):
        rows_v[0][i, pl.ds(r * _L, _L)] = zero16
      return 0
    lax.fori_loop(0, _CH, zrow, 0)
    r0 = sid * RPT
    nfull = RPT // _CH
    rrem = RPT % _CH

    def zcp(k, _):
      pltpu.sync_copy(rows_v[0], out_sp.at[pl.ds(r0 + k * _CH, _CH)])
      return 0
    lax.fori_loop(0, nfull, zcp, 0)
    if rrem:
      pltpu.sync_copy(rows_v[0].at[pl.ds(0, rrem)],
                      out_sp.at[pl.ds(r0 + nfull * _CH, rrem)])

    @pl.when(sid == _NS - 1)
    def _():
      pltpu.sync_copy(rows_v[0].at[pl.ds(0, TAIL)],
                      out_sp.at[pl.ds(N - TAIL, TAIL)])

    @pl.when(sid < N // ZC)
    def _():
      def zden(i, _):
        den_stage[pl.ds(i * _L, _L)] = zero16
        return 0
      lax.fori_loop(0, ZC // _L, zden, 0)
      pltpu.sync_copy(den_stage, den_sp.at[pl.ds(sid * ZC, ZC)])

    plsc.subcore_barrier()

    idx_row0 = wid * (EP // _SS)
    assert n_chunks % _SB == 0
    n_super = n_chunks // _SB
    iota16 = lax.iota(jnp.int32, _L)

    def fire(b, j):
      pltpu.async_copy(hs_hbm.at[src_big.at[j]], rows_v[b], sem[b])
      pltpu.async_copy(als_hbm.at[src_big.at[j]], als_g[b], sem[b])
      pltpu.async_copy(ald_hbm.at[dst_big.at[j]], ald_g[b], sem[b])

    def wait_fired(b):
      pltpu.make_async_copy(hs_hbm.at[src_big.at[0]], rows_v[b],
                            sem[b]).wait()
      pltpu.make_async_copy(als_hbm.at[src_big.at[0]], als_g[b],
                            sem[b]).wait()
      pltpu.make_async_copy(ald_hbm.at[dst_big.at[0]], ald_g[b],
                            sem[b]).wait()

    def compute_and_scatter(b, j):
      def grp(g, _):
        sl = pl.ds(g * _L, _L)
        e16 = als_g[b][sl] + ald_g[b][sl]
        e16 = jnp.where(e16 >= 0.0, e16, 0.2 * e16)
        ex16 = jnp.exp(e16)
        ex_v[b][sl] = ex16
        ridx = iota16 + g * _L
        for r in range(8):
          cr = jnp.full((_L,), r, jnp.int32)
          v = plsc.load_gather(rows_v[b], [ridx, cr])
          plsc.store_scatter(rows_v[b], [ridx, cr], v * ex16)
        return 0
      lax.fori_loop(0, _SS // _L, grp, 0)
      pltpu.sync_copy(rows_v[b], out_sp.at[dst_big.at[j]], add=True)
      pltpu.sync_copy(ex_v[b], den_sp.at[dst_big.at[j]], add=True)

    def super_body(s, _):
      srow = idx_row0 + s * _SB
      pltpu.sync_copy(src_hbm.at[pl.ds(srow, _SB)], src_big)
      pltpu.sync_copy(dst_hbm.at[pl.ds(srow, _SB)], dst_big)

      fire(0, 0)

      def pair_body(i, _):
        j0 = 2 * i

        @pl.when(j0 + 1 < _SB)
        def _():
          fire(1, j0 + 1)
        wait_fired(0)
        compute_and_scatter(0, j0)

        @pl.when(j0 + 1 < _SB)
        def _():
          @pl.when(j0 + 2 < _SB)
          def _():
            fire(0, j0 + 2)
          wait_fired(1)
          compute_and_scatter(1, j0 + 1)
        return 0

      lax.fori_loop(0, (_SB + 1) // 2, pair_body, 0)
      return 0

    lax.fori_loop(0, n_super, super_body, 0)

    plsc.subcore_barrier()

    def ocp(k, _):
      pltpu.sync_copy(out_sp.at[pl.ds(r0 + k * _CH, _CH)], rows_v[0])
      pltpu.sync_copy(rows_v[0], out_hbm.at[cid, pl.ds(r0 + k * _CH, _CH)])
      return 0
    lax.fori_loop(0, nfull, ocp, 0)
    if rrem:
      pltpu.sync_copy(out_sp.at[pl.ds(r0 + nfull * _CH, rrem)],
                      rows_v[0].at[pl.ds(0, rrem)])
      pltpu.sync_copy(rows_v[0].at[pl.ds(0, rrem)],
                      out_hbm.at[cid, pl.ds(r0 + nfull * _CH, rrem)])

    @pl.when(sid == _NS - 1)
    def _():
      pltpu.sync_copy(out_sp.at[pl.ds(N - TAIL, TAIL)],
                      rows_v[1].at[pl.ds(0, TAIL)])
      pltpu.sync_copy(rows_v[1].at[pl.ds(0, TAIL)],
                      out_hbm.at[cid, pl.ds(N - TAIL, TAIL)])

    @pl.when(sid < N // ZC)
    def _():
      pltpu.sync_copy(den_sp.at[pl.ds(sid * ZC, ZC)], den_stage)
      pltpu.sync_copy(den_stage, den_hbm.at[cid, pl.ds(sid * ZC, ZC)])

  return sc_kernel


_B = 1000


def _tc1_body(x, wsrc, wdst, a_s, a_d, hs_o, als_o, ald_o):
  xb = x[...]
  hs = jnp.dot(xb, wsrc[...], preferred_element_type=jnp.float32)
  hs_o[...] = hs
  als_o[...] = jnp.dot(hs, a_s[...], preferred_element_type=jnp.float32)
  vd = jnp.dot(wdst[...], a_d[...], preferred_element_type=jnp.float32)
  ald_o[...] = jnp.dot(xb, vd, preferred_element_type=jnp.float32)


def _tc2_body(outA, outB, denA, denB, x, b0, wl0, bl0, w1, as1, ad1,
              h_o, hs1_o, als1_o, ald1_o):
  d = denA[...] + denB[...] + 1e-16
  gat = (outA[...] + outB[...]) / d
  xb = x[...]
  h = jnp.maximum(
      gat + b0[...]
      + jnp.dot(xb, wl0[...], preferred_element_type=jnp.float32)
      + bl0[...], 0.0)
  h_o[...] = h
  hs1 = jnp.dot(h, w1[...], preferred_element_type=jnp.float32)
  hs1_o[...] = hs1
  als1_o[...] = jnp.dot(hs1, as1[...], preferred_element_type=jnp.float32)
  vd1 = jnp.dot(w1[...], ad1[...], preferred_element_type=jnp.float32)
  ald1_o[...] = jnp.dot(h, vd1, preferred_element_type=jnp.float32)


def _tc3_body(outA, outB, denA, denB, h, b1, wl1, bl1,
              h2_o, ssum_o, sqsum_o):
  d = denA[...] + denB[...] + 1e-16
  gat = (outA[...] + outB[...]) / d
  h2 = (gat + b1[...]
        + jnp.dot(h[...], wl1[...], preferred_element_type=jnp.float32)
        + bl1[...])
  h2_o[...] = h2
  i = pl.program_id(0)
  ssum_o[pl.ds(i, 1), :] = jnp.sum(h2, axis=0, keepdims=True)
  sqsum_o[pl.ds(i, 1), :] = jnp.sum(h2 * h2, axis=0, keepdims=True)


def _tc4_body(nd, h2, ssum, sqsum, ln_w, ln_b, wp, bp, y_o):
  mu = jnp.sum(ssum[...]) / nd
  var = jnp.sum(sqsum[...]) / nd - mu * mu
  inv = lax.rsqrt(var + 1e-5)
  hn = (h2[...] - mu) * inv * ln_w[...] + ln_b[...]
  y_o[...] = jnp.dot(hn, wp[...], preferred_element_type=jnp.float32) + bp[...]


def kernel(x, edge_index, w_src0, w_dst0, a_src0, a_dst0, b0, wl0, bl0,
           w1, a_src1, a_dst1, b1, wl1, bl1, ln_w, ln_b, wp, bp):
  N, D = x.shape
  E = edge_index.shape[1]
  nb = N // _B

  src2 = edge_index[0].reshape(E // _SS, _SS)
  dst2 = edge_index[1].reshape(E // _SS, _SS)

  a_s0c = a_src0.reshape(D, 1)
  a_d0c = a_dst0.reshape(D, 1)
  a_s1c = a_src1.reshape(D, 1)
  a_d1c = a_dst1.reshape(D, 1)
  b0r = b0.reshape(1, D)
  bl0r = bl0.reshape(1, D)
  b1r = b1.reshape(1, D)
  bl1r = bl1.reshape(1, D)
  ln_wr = ln_w.reshape(1, D)
  ln_br = ln_b.reshape(1, D)
  bpr = bp.reshape(1, 1)

  row_spec = pl.BlockSpec((_B, D), lambda i: (i, 0))
  col1_spec = pl.BlockSpec((_B, 1), lambda i: (i, 0))
  w_spec = pl.BlockSpec((D, D), lambda i: (0, 0))
  v_spec = pl.BlockSpec((D, 1), lambda i: (0, 0))
  r_spec = pl.BlockSpec((1, D), lambda i: (0, 0))

  hs0, als0, ald0 = pl.pallas_call(
      _tc1_body,
      grid=(nb,),
      in_specs=[row_spec, w_spec, w_spec, v_spec, v_spec],
      out_specs=[row_spec, col1_spec, col1_spec],
      out_shape=[
          jax.ShapeDtypeStruct((N, D), jnp.float32),
          jax.ShapeDtypeStruct((N, 1), jnp.float32),
          jax.ShapeDtypeStruct((N, 1), jnp.float32),
      ],
  )(x, w_src0, w_dst0, a_s0c, a_d0c)

  sc_edge = _make_sc_edge_kernel(N, E)

  out2_0, den2_0 = sc_edge(hs0, als0.reshape(N), ald0.reshape(N),
                           src2, dst2)

  h, hs1, als1, ald1 = pl.pallas_call(
      _tc2_body,
      grid=(nb,),
      in_specs=[row_spec, row_spec, col1_spec, col1_spec, row_spec,
                r_spec, w_spec, r_spec, w_spec, v_spec, v_spec],
      out_specs=[row_spec, row_spec, col1_spec, col1_spec],
      out_shape=[
          jax.ShapeDtypeStruct((N, D), jnp.float32),
          jax.ShapeDtypeStruct((N, D), jnp.float32),
          jax.ShapeDtypeStruct((N, 1), jnp.float32),
          jax.ShapeDtypeStruct((N, 1), jnp.float32),
      ],
  )(out2_0[0], out2_0[1], den2_0[0].reshape(N, 1), den2_0[1].reshape(N, 1),
    x, b0r, wl0, bl0r, w1, a_s1c, a_d1c)

  out2_1, den2_1 = sc_edge(hs1, als1.reshape(N), ald1.reshape(N),
                           src2, dst2)

  h2, ssum, sqsum = pl.pallas_call(
      _tc3_body,
      grid=(nb,),
      in_specs=[row_spec, row_spec, col1_spec, col1_spec, row_spec,
                r_spec, w_spec, r_spec],
      out_specs=[row_spec, pl.BlockSpec((nb, D), lambda i: (0, 0)),
                 pl.BlockSpec((nb, D), lambda i: (0, 0))],
      out_shape=[
          jax.ShapeDtypeStruct((N, D), jnp.float32),
          jax.ShapeDtypeStruct((nb, D), jnp.float32),
          jax.ShapeDtypeStruct((nb, D), jnp.float32),
      ],
  )(out2_1[0], out2_1[1], den2_1[0].reshape(N, 1), den2_1[1].reshape(N, 1),
    h, b1r, wl1, bl1r)

  y = pl.pallas_call(
      functools.partial(_tc4_body, float(N * D)),
      grid=(nb,),
      in_specs=[row_spec, pl.BlockSpec((nb, D), lambda i: (0, 0)),
                pl.BlockSpec((nb, D), lambda i: (0, 0)),
                r_spec, r_spec, v_spec, pl.BlockSpec((1, 1), lambda i: (0, 0))],
      out_specs=[col1_spec],
      out_shape=[jax.ShapeDtypeStruct((N, 1), jnp.float32)],
  )(h2, ssum, sqsum, ln_wr, ln_br, wp, bpr)[0]

  return y


# speedup vs baseline: 43.4322x; 1.1520x over previous
import functools

import jax
import jax.numpy as jnp
from jax import lax
from jax.experimental import pallas as pl
from jax.experimental.pallas import tpu as pltpu
from jax.experimental.pallas import tpu_sc as plsc

_NC = 2
_NS = 16
_NW = _NC * _NS
_L = 16

_SS = 80
_JROWS = 1
_CH = _SS * _JROWS


_SB = 25


def _splat(vec, j):
  idx = jnp.full((_L, 1), j, jnp.int32)
  dn = lax.GatherDimensionNumbers(
      offset_dims=(), collapsed_slice_dims=(0,), start_index_map=(0,))
  return lax.gather(vec, idx, dn, (1,),
                    mode=lax.GatherScatterMode.PROMISE_IN_BOUNDS)


def _make_sc_edge_kernel(n_nodes, n_edges):
  N = n_nodes
  EP = n_edges // _NW
  assert EP % _CH == 0
  n_chunks = EP // _CH
  RPT = (N // _NS) & ~7
  TAIL = N - _NS * RPT
  ZC = 2000

  mesh = plsc.VectorSubcoreMesh(core_axis_name="c", subcore_axis_name="s",
                                num_cores=_NC, num_subcores=_NS)

  @functools.partial(
      pl.kernel,
      out_type=(
          jax.ShapeDtypeStruct((_NC, N, 128), jnp.float32),
          jax.ShapeDtypeStruct((_NC, N), jnp.float32),
      ),
      mesh=mesh,
      scratch_types=dict(
          src_big=pltpu.VMEM((_SB, _SS), jnp.int32),
          dst_big=pltpu.VMEM((_SB, _SS), jnp.int32),
          als_v=pltpu.VMEM((N,), jnp.float32),
          ald_v=pltpu.VMEM((N,), jnp.float32),
          ex_v=tuple(pltpu.VMEM((_SS,), jnp.float32) for _ in range(2)),
          rows_v=tuple(pltpu.VMEM((_CH, 128), jnp.float32) for _ in range(2)),
          den_stage=pltpu.VMEM((ZC,), jnp.float32),
          out_sp=pltpu.VMEM_SHARED((N, 128), jnp.float32),
          den_sp=pltpu.VMEM_SHARED((N,), jnp.float32),
          sem=tuple(pltpu.SemaphoreType.DMA for _ in range(2)),
      ),
      compiler_params=pltpu.CompilerParams(use_tc_tiling_on_sc=False,
                                           needs_layout_passes=False),
  )
  def sc_kernel(hs_hbm, als_hbm, ald_hbm, src_hbm, dst_hbm,
                out_hbm, den_hbm, *, src_big, dst_big, als_v, ald_v, ex_v,
                rows_v, den_stage, out_sp, den_sp, sem):
    cid = lax.axis_index("c")
    sid = lax.axis_index("s")
    wid = sid * _NC + cid

    zero16 = jnp.zeros((_L,), jnp.float32)

    pltpu.sync_copy(als_hbm, als_v)
    pltpu.sync_copy(ald_hbm, ald_v)

    def zrow(i, _):
      for r in range(8):
        rows_v[0][i, pl.ds(r * _L, _L)] = zero16
      return 0
    lax.fori_loop(0, _CH, zrow, 0)
    r0 = sid * RPT
    nfull = RPT // _CH
    rrem = RPT % _CH

    def zcp(k, _):
      pltpu.sync_copy(rows_v[0], out_sp.at[pl.ds(r0 + k * _CH, _CH)])
      return 0
    lax.fori_loop(0, nfull, zcp, 0)
    if rrem:
      pltpu.sync_copy(rows_v[0].at[pl.ds(0, rrem)],
                      out_sp.at[pl.ds(r0 + nfull * _CH, rrem)])

    @pl.when(sid == _NS - 1)
    def _():
      pltpu.sync_copy(rows_v[0].at[pl.ds(0, TAIL)],
                      out_sp.at[pl.ds(N - TAIL, TAIL)])

    @pl.when(sid < N // ZC)
    def _():
      def zden(i, _):
        den_stage[pl.ds(i * _L, _L)] = zero16
        return 0
      lax.fori_loop(0, ZC // _L, zden, 0)
      pltpu.sync_copy(den_stage, den_sp.at[pl.ds(sid * ZC, ZC)])

    plsc.subcore_barrier()

    idx_row0 = wid * (EP // _SS)
    assert n_chunks % _SB == 0
    n_super = n_chunks // _SB

    def fire(b, j):
      pltpu.async_copy(hs_hbm.at[src_big.at[j]], rows_v[b], sem[b])

    def wait_fired(b):
      pltpu.make_async_copy(hs_hbm.at[src_big.at[0]], rows_v[b],
                            sem[b]).wait()

    def compute_and_scatter(b, j):
      def grp(g, _):
        sl = pl.ds(g * _L, _L)
        s16 = src_big[j, sl]
        d16 = dst_big[j, sl]
        e16 = plsc.load_gather(als_v, [s16]) + plsc.load_gather(ald_v, [d16])
        e16 = jnp.where(e16 >= 0.0, e16, 0.2 * e16)
        ex16 = jnp.exp(e16)
        ex_v[b][sl] = ex16
        for jj in range(_L):
          e0 = g * _L + jj
          sp = _splat(ex16, jj)
          for r in range(8):
            rsl = pl.ds(r * _L, _L)
            rows_v[b][e0, rsl] = rows_v[b][e0, rsl] * sp
        return 0
      lax.fori_loop(0, _SS // _L, grp, 0)
      pltpu.sync_copy(rows_v[b], out_sp.at[dst_big.at[j]], add=True)
      pltpu.sync_copy(ex_v[b], den_sp.at[dst_big.at[j]], add=True)

    def super_body(s, _):
      srow = idx_row0 + s * _SB
      pltpu.sync_copy(src_hbm.at[pl.ds(srow, _SB)], src_big)
      pltpu.sync_copy(dst_hbm.at[pl.ds(srow, _SB)], dst_big)

      fire(0, 0)

      def pair_body(i, _):
        j0 = 2 * i

        @pl.when(j0 + 1 < _SB)
        def _():
          fire(1, j0 + 1)
        wait_fired(0)
        compute_and_scatter(0, j0)

        @pl.when(j0 + 1 < _SB)
        def _():
          @pl.when(j0 + 2 < _SB)
          def _():
            fire(0, j0 + 2)
          wait_fired(1)
          compute_and_scatter(1, j0 + 1)
        return 0

      lax.fori_loop(0, (_SB + 1) // 2, pair_body, 0)
      return 0

    lax.fori_loop(0, n_super, super_body, 0)

    plsc.subcore_barrier()

    def ocp(k, _):
      pltpu.sync_copy(out_sp.at[pl.ds(r0 + k * _CH, _CH)], rows_v[0])
      pltpu.sync_copy(rows_v[0], out_hbm.at[cid, pl.ds(r0 + k * _CH, _CH)])
      return 0
    lax.fori_loop(0, nfull, ocp, 0)
    if rrem:
      pltpu.sync_copy(out_sp.at[pl.ds(r0 + nfull * _CH, rrem)],
                      rows_v[0].at[pl.ds(0, rrem)])
      pltpu.sync_copy(rows_v[0].at[pl.ds(0, rrem)],
                      out_hbm.at[cid, pl.ds(r0 + nfull * _CH, rrem)])

    @pl.when(sid == _NS - 1)
    def _():
      pltpu.sync_copy(out_sp.at[pl.ds(N - TAIL, TAIL)],
                      rows_v[1].at[pl.ds(0, TAIL)])
      pltpu.sync_copy(rows_v[1].at[pl.ds(0, TAIL)],
                      out_hbm.at[cid, pl.ds(N - TAIL, TAIL)])

    @pl.when(sid < N // ZC)
    def _():
      pltpu.sync_copy(den_sp.at[pl.ds(sid * ZC, ZC)], den_stage)
      pltpu.sync_copy(den_stage, den_hbm.at[cid, pl.ds(sid * ZC, ZC)])

  return sc_kernel


_B = 1000


def _tc1_body(x, wsrc, wdst, a_s, a_d, hs_o, als_o, ald_o):
  xb = x[...]
  hs = jnp.dot(xb, wsrc[...], preferred_element_type=jnp.float32)
  hs_o[...] = hs
  als_o[...] = jnp.dot(hs, a_s[...], preferred_element_type=jnp.float32)
  vd = jnp.dot(wdst[...], a_d[...], preferred_element_type=jnp.float32)
  ald_o[...] = jnp.dot(xb, vd, preferred_element_type=jnp.float32)


def _tc2_body(outA, outB, denA, denB, x, b0, wl0, bl0, w1, as1, ad1,
              h_o, hs1_o, als1_o, ald1_o):
  d = denA[...] + denB[...] + 1e-16
  gat = (outA[...] + outB[...]) / d
  xb = x[...]
  h = jnp.maximum(
      gat + b0[...]
      + jnp.dot(xb, wl0[...], preferred_element_type=jnp.float32)
      + bl0[...], 0.0)
  h_o[...] = h
  hs1 = jnp.dot(h, w1[...], preferred_element_type=jnp.float32)
  hs1_o[...] = hs1
  als1_o[...] = jnp.dot(hs1, as1[...], preferred_element_type=jnp.float32)
  vd1 = jnp.dot(w1[...], ad1[...], preferred_element_type=jnp.float32)
  ald1_o[...] = jnp.dot(h, vd1, preferred_element_type=jnp.float32)


def _tc3_body(outA, outB, denA, denB, h, b1, wl1, bl1,
              h2_o, ssum_o, sqsum_o):
  d = denA[...] + denB[...] + 1e-16
  gat = (outA[...] + outB[...]) / d
  h2 = (gat + b1[...]
        + jnp.dot(h[...], wl1[...], preferred_element_type=jnp.float32)
        + bl1[...])
  h2_o[...] = h2
  i = pl.program_id(0)
  ssum_o[pl.ds(i, 1), :] = jnp.sum(h2, axis=0, keepdims=True)
  sqsum_o[pl.ds(i, 1), :] = jnp.sum(h2 * h2, axis=0, keepdims=True)


def _tc4_body(nd, h2, ssum, sqsum, ln_w, ln_b, wp, bp, y_o):
  mu = jnp.sum(ssum[...]) / nd
  var = jnp.sum(sqsum[...]) / nd - mu * mu
  inv = lax.rsqrt(var + 1e-5)
  hn = (h2[...] - mu) * inv * ln_w[...] + ln_b[...]
  y_o[...] = jnp.dot(hn, wp[...], preferred_element_type=jnp.float32) + bp[...]


def kernel(x, edge_index, w_src0, w_dst0, a_src0, a_dst0, b0, wl0, bl0,
           w1, a_src1, a_dst1, b1, wl1, bl1, ln_w, ln_b, wp, bp):
  N, D = x.shape
  E = edge_index.shape[1]
  nb = N // _B

  src2 = edge_index[0].reshape(E // _SS, _SS)
  dst2 = edge_index[1].reshape(E // _SS, _SS)

  a_s0c = a_src0.reshape(D, 1)
  a_d0c = a_dst0.reshape(D, 1)
  a_s1c = a_src1.reshape(D, 1)
  a_d1c = a_dst1.reshape(D, 1)
  b0r = b0.reshape(1, D)
  bl0r = bl0.reshape(1, D)
  b1r = b1.reshape(1, D)
  bl1r = bl1.reshape(1, D)
  ln_wr = ln_w.reshape(1, D)
  ln_br = ln_b.reshape(1, D)
  bpr = bp.reshape(1, 1)

  row_spec = pl.BlockSpec((_B, D), lambda i: (i, 0))
  col1_spec = pl.BlockSpec((_B, 1), lambda i: (i, 0))
  w_spec = pl.BlockSpec((D, D), lambda i: (0, 0))
  v_spec = pl.BlockSpec((D, 1), lambda i: (0, 0))
  r_spec = pl.BlockSpec((1, D), lambda i: (0, 0))

  hs0, als0, ald0 = pl.pallas_call(
      _tc1_body,
      grid=(nb,),
      in_specs=[row_spec, w_spec, w_spec, v_spec, v_spec],
      out_specs=[row_spec, col1_spec, col1_spec],
      out_shape=[
          jax.ShapeDtypeStruct((N, D), jnp.float32),
          jax.ShapeDtypeStruct((N, 1), jnp.float32),
          jax.ShapeDtypeStruct((N, 1), jnp.float32),
      ],
  )(x, w_src0, w_dst0, a_s0c, a_d0c)

  sc_edge = _make_sc_edge_kernel(N, E)

  out2_0, den2_0 = sc_edge(hs0, als0.reshape(N), ald0.reshape(N),
                           src2, dst2)

  h, hs1, als1, ald1 = pl.pallas_call(
      _tc2_body,
      grid=(nb,),
      in_specs=[row_spec, row_spec, col1_spec, col1_spec, row_spec,
                r_spec, w_spec, r_spec, w_spec, v_spec, v_spec],
      out_specs=[row_spec, row_spec, col1_spec, col1_spec],
      out_shape=[
          jax.ShapeDtypeStruct((N, D), jnp.float32),
          jax.ShapeDtypeStruct((N, D), jnp.float32),
          jax.ShapeDtypeStruct((N, 1), jnp.float32),
          jax.ShapeDtypeStruct((N, 1), jnp.float32),
      ],
  )(out2_0[0], out2_0[1], den2_0[0].reshape(N, 1), den2_0[1].reshape(N, 1),
    x, b0r, wl0, bl0r, w1, a_s1c, a_d1c)

  out2_1, den2_1 = sc_edge(hs1, als1.reshape(N), ald1.reshape(N),
                           src2, dst2)

  h2, ssum, sqsum = pl.pallas_call(
      _tc3_body,
      grid=(nb,),
      in_specs=[row_spec, row_spec, col1_spec, col1_spec, row_spec,
                r_spec, w_spec, r_spec],
      out_specs=[row_spec, pl.BlockSpec((nb, D), lambda i: (0, 0)),
                 pl.BlockSpec((nb, D), lambda i: (0, 0))],
      out_shape=[
          jax.ShapeDtypeStruct((N, D), jnp.float32),
          jax.ShapeDtypeStruct((nb, D), jnp.float32),
          jax.ShapeDtypeStruct((nb, D), jnp.float32),
      ],
  )(out2_1[0], out2_1[1], den2_1[0].reshape(N, 1), den2_1[1].reshape(N, 1),
    h, b1r, wl1, bl1r)

  y = pl.pallas_call(
      functools.partial(_tc4_body, float(N * D)),
      grid=(nb,),
      in_specs=[row_spec, pl.BlockSpec((nb, D), lambda i: (0, 0)),
                pl.BlockSpec((nb, D), lambda i: (0, 0)),
                r_spec, r_spec, v_spec, pl.BlockSpec((1, 1), lambda i: (0, 0))],
      out_specs=[col1_spec],
      out_shape=[jax.ShapeDtypeStruct((N, 1), jnp.float32)],
  )(h2, ssum, sqsum, ln_wr, ln_br, wp, bpr)[0]

  return y


# speedup vs baseline: 47.2195x; 1.0872x over previous
import functools

import jax
import jax.numpy as jnp
from jax import lax
from jax.experimental import pallas as pl
from jax.experimental.pallas import tpu as pltpu
from jax.experimental.pallas import tpu_sc as plsc

_NC = 2
_NS = 16
_NW = _NC * _NS
_L = 16

_SS = 80
_JROWS = 1
_CH = _SS * _JROWS


_SB = 25


def _splat(vec, j):
  idx = jnp.full((_L, 1), j, jnp.int32)
  dn = lax.GatherDimensionNumbers(
      offset_dims=(), collapsed_slice_dims=(0,), start_index_map=(0,))
  return lax.gather(vec, idx, dn, (1,),
                    mode=lax.GatherScatterMode.PROMISE_IN_BOUNDS)


def _make_sc_edge_kernel(n_nodes, n_edges):
  N = n_nodes
  EP = n_edges // _NW
  assert EP % _CH == 0
  n_chunks = EP // _CH
  RPT = (N // _NS) & ~7
  TAIL = N - _NS * RPT
  ZC = 2000

  mesh = plsc.VectorSubcoreMesh(core_axis_name="c", subcore_axis_name="s",
                                num_cores=_NC, num_subcores=_NS)

  @functools.partial(
      pl.kernel,
      out_type=(
          jax.ShapeDtypeStruct((_NC, N, 128), jnp.float32),
          jax.ShapeDtypeStruct((_NC, N), jnp.float32),
      ),
      mesh=mesh,
      scratch_types=dict(
          src_big=pltpu.VMEM((_SB, _SS), jnp.int32),
          dst_big=pltpu.VMEM((_SB, _SS), jnp.int32),
          als_g=tuple(pltpu.VMEM((_SS,), jnp.float32) for _ in range(3)),
          ald_g=tuple(pltpu.VMEM((_SS,), jnp.float32) for _ in range(3)),
          ex_v=tuple(pltpu.VMEM((_SS,), jnp.float32) for _ in range(3)),
          rows_v=tuple(pltpu.VMEM((_CH, 128), jnp.float32) for _ in range(3)),
          den_stage=pltpu.VMEM((ZC,), jnp.float32),
          out_sp=pltpu.VMEM_SHARED((N, 128), jnp.float32),
          den_sp=pltpu.VMEM_SHARED((N,), jnp.float32),
          sem=tuple(pltpu.SemaphoreType.DMA for _ in range(3)),
          sem_s=tuple(pltpu.SemaphoreType.DMA for _ in range(3)),
      ),
      compiler_params=pltpu.CompilerParams(use_tc_tiling_on_sc=False,
                                           needs_layout_passes=False),
  )
  def sc_kernel(hs_hbm, als_hbm, ald_hbm, src_hbm, dst_hbm,
                out_hbm, den_hbm, *, src_big, dst_big, als_g, ald_g, ex_v,
                rows_v, den_stage, out_sp, den_sp, sem, sem_s):
    cid = lax.axis_index("c")
    sid = lax.axis_index("s")
    wid = sid * _NC + cid

    zero16 = jnp.zeros((_L,), jnp.float32)

    def zrow(i, _):
      for r in range(8):
        rows_v[0][i, pl.ds(r * _L, _L)] = zero16
      return 0
    lax.fori_loop(0, _CH, zrow, 0)
    r0 = sid * RPT
    nfull = RPT // _CH
    rrem = RPT % _CH

    def zcp(k, _):
      pltpu.sync_copy(rows_v[0], out_sp.at[pl.ds(r0 + k * _CH, _CH)])
      return 0
    lax.fori_loop(0, nfull, zcp, 0)
    if rrem:
      pltpu.sync_copy(rows_v[0].at[pl.ds(0, rrem)],
                      out_sp.at[pl.ds(r0 + nfull * _CH, rrem)])

    @pl.when(sid == _NS - 1)
    def _():
      pltpu.sync_copy(rows_v[0].at[pl.ds(0, TAIL)],
                      out_sp.at[pl.ds(N - TAIL, TAIL)])

    @pl.when(sid < N // ZC)
    def _():
      def zden(i, _):
        den_stage[pl.ds(i * _L, _L)] = zero16
        return 0
      lax.fori_loop(0, ZC // _L, zden, 0)
      pltpu.sync_copy(den_stage, den_sp.at[pl.ds(sid * ZC, ZC)])

    plsc.subcore_barrier()

    idx_row0 = wid * (EP // _SS)
    assert n_chunks % _SB == 0
    n_super = n_chunks // _SB

    def fire(b, j):
      pltpu.async_copy(hs_hbm.at[src_big.at[j]], rows_v[b], sem[b])
      pltpu.async_copy(als_hbm.at[src_big.at[j]], als_g[b], sem[b])
      pltpu.async_copy(ald_hbm.at[dst_big.at[j]], ald_g[b], sem[b])

    def wait_fired(b):
      pltpu.make_async_copy(hs_hbm.at[src_big.at[0]], rows_v[b],
                            sem[b]).wait()
      pltpu.make_async_copy(als_hbm.at[src_big.at[0]], als_g[b],
                            sem[b]).wait()
      pltpu.make_async_copy(ald_hbm.at[dst_big.at[0]], ald_g[b],
                            sem[b]).wait()

    def compute(b):
      def grp(g, _):
        sl = pl.ds(g * _L, _L)
        e16 = als_g[b][sl] + ald_g[b][sl]
        e16 = jnp.where(e16 >= 0.0, e16, 0.2 * e16)
        ex16 = jnp.exp(e16)
        ex_v[b][sl] = ex16
        for jj in range(_L):
          e0 = g * _L + jj
          sp = _splat(ex16, jj)
          for r in range(8):
            rsl = pl.ds(r * _L, _L)
            rows_v[b][e0, rsl] = rows_v[b][e0, rsl] * sp
        return 0
      lax.fori_loop(0, _SS // _L, grp, 0)

    def fire_scatter(b, j):
      pltpu.async_copy(rows_v[b], out_sp.at[dst_big.at[j]], sem_s[b],
                       add=True)
      pltpu.async_copy(ex_v[b], den_sp.at[dst_big.at[j]], sem_s[b],
                       add=True)

    def wait_scatter(b):
      pltpu.make_async_copy(rows_v[b], out_sp.at[dst_big.at[0]],
                            sem_s[b]).wait()
      pltpu.make_async_copy(ex_v[b], den_sp.at[dst_big.at[0]],
                            sem_s[b]).wait()

    def super_body(s, _):
      srow = idx_row0 + s * _SB
      pltpu.sync_copy(src_hbm.at[pl.ds(srow, _SB)], src_big)
      pltpu.sync_copy(dst_hbm.at[pl.ds(srow, _SB)], dst_big)

      fire(0, 0)
      fire(1, 1)

      def tri_body(i, _):
        for t in range(3):
          j = 3 * i + t

          @pl.when(j < _SB)
          def _(t=t, j=j):
            wait_fired(t)
            compute(t)

            @pl.when(j >= 1)
            def _():
              wait_scatter((t + 2) % 3)

            @pl.when(j + 2 < _SB)
            def _():
              fire((t + 2) % 3, j + 2)
            fire_scatter(t, j)
        return 0

      lax.fori_loop(0, (_SB + 2) // 3, tri_body, 0)
      wait_scatter((_SB - 1) % 3)
      return 0

    lax.fori_loop(0, n_super, super_body, 0)

    plsc.subcore_barrier()

    def ocp(k, _):
      pltpu.sync_copy(out_sp.at[pl.ds(r0 + k * _CH, _CH)], rows_v[0])
      pltpu.sync_copy(rows_v[0], out_hbm.at[cid, pl.ds(r0 + k * _CH, _CH)])
      return 0
    lax.fori_loop(0, nfull, ocp, 0)
    if rrem:
      pltpu.sync_copy(out_sp.at[pl.ds(r0 + nfull * _CH, rrem)],
                      rows_v[0].at[pl.ds(0, rrem)])
      pltpu.sync_copy(rows_v[0].at[pl.ds(0, rrem)],
                      out_hbm.at[cid, pl.ds(r0 + nfull * _CH, rrem)])

    @pl.when(sid == _NS - 1)
    def _():
      pltpu.sync_copy(out_sp.at[pl.ds(N - TAIL, TAIL)],
                      rows_v[1].at[pl.ds(0, TAIL)])
      pltpu.sync_copy(rows_v[1].at[pl.ds(0, TAIL)],
                      out_hbm.at[cid, pl.ds(N - TAIL, TAIL)])

    @pl.when(sid < N // ZC)
    def _():
      pltpu.sync_copy(den_sp.at[pl.ds(sid * ZC, ZC)], den_stage)
      pltpu.sync_copy(den_stage, den_hbm.at[cid, pl.ds(sid * ZC, ZC)])

  return sc_kernel


_B = 1000


def _tc1_body(x, wsrc, wdst, a_s, a_d, hs_o, als_o, ald_o):
  xb = x[...]
  hs = jnp.dot(xb, wsrc[...], preferred_element_type=jnp.float32)
  hs_o[...] = hs
  als_o[...] = jnp.dot(hs, a_s[...], preferred_element_type=jnp.float32)
  vd = jnp.dot(wdst[...], a_d[...], preferred_element_type=jnp.float32)
  ald_o[...] = jnp.dot(xb, vd, preferred_element_type=jnp.float32)


def _tc2_body(outA, outB, denA, denB, x, b0, wl0, bl0, w1, as1, ad1,
              h_o, hs1_o, als1_o, ald1_o):
  d = denA[...] + denB[...] + 1e-16
  gat = (outA[...] + outB[...]) / d
  xb = x[...]
  h = jnp.maximum(
      gat + b0[...]
      + jnp.dot(xb, wl0[...], preferred_element_type=jnp.float32)
      + bl0[...], 0.0)
  h_o[...] = h
  hs1 = jnp.dot(h, w1[...], preferred_element_type=jnp.float32)
  hs1_o[...] = hs1
  als1_o[...] = jnp.dot(hs1, as1[...], preferred_element_type=jnp.float32)
  vd1 = jnp.dot(w1[...], ad1[...], preferred_element_type=jnp.float32)
  ald1_o[...] = jnp.dot(h, vd1, preferred_element_type=jnp.float32)


def _tc3_body(outA, outB, denA, denB, h, b1, wl1, bl1,
              h2_o, ssum_o, sqsum_o):
  d = denA[...] + denB[...] + 1e-16
  gat = (outA[...] + outB[...]) / d
  h2 = (gat + b1[...]
        + jnp.dot(h[...], wl1[...], preferred_element_type=jnp.float32)
        + bl1[...])
  h2_o[...] = h2
  i = pl.program_id(0)
  ssum_o[pl.ds(i, 1), :] = jnp.sum(h2, axis=0, keepdims=True)
  sqsum_o[pl.ds(i, 1), :] = jnp.sum(h2 * h2, axis=0, keepdims=True)


def _tc4_body(nd, h2, ssum, sqsum, ln_w, ln_b, wp, bp, y_o):
  mu = jnp.sum(ssum[...]) / nd
  var = jnp.sum(sqsum[...]) / nd - mu * mu
  inv = lax.rsqrt(var + 1e-5)
  hn = (h2[...] - mu) * inv * ln_w[...] + ln_b[...]
  y_o[...] = jnp.dot(hn, wp[...], preferred_element_type=jnp.float32) + bp[...]


def kernel(x, edge_index, w_src0, w_dst0, a_src0, a_dst0, b0, wl0, bl0,
           w1, a_src1, a_dst1, b1, wl1, bl1, ln_w, ln_b, wp, bp):
  N, D = x.shape
  E = edge_index.shape[1]
  nb = N // _B

  src2 = edge_index[0].reshape(E // _SS, _SS)
  dst2 = edge_index[1].reshape(E // _SS, _SS)

  a_s0c = a_src0.reshape(D, 1)
  a_d0c = a_dst0.reshape(D, 1)
  a_s1c = a_src1.reshape(D, 1)
  a_d1c = a_dst1.reshape(D, 1)
  b0r = b0.reshape(1, D)
  bl0r = bl0.reshape(1, D)
  b1r = b1.reshape(1, D)
  bl1r = bl1.reshape(1, D)
  ln_wr = ln_w.reshape(1, D)
  ln_br = ln_b.reshape(1, D)
  bpr = bp.reshape(1, 1)

  row_spec = pl.BlockSpec((_B, D), lambda i: (i, 0))
  col1_spec = pl.BlockSpec((_B, 1), lambda i: (i, 0))
  w_spec = pl.BlockSpec((D, D), lambda i: (0, 0))
  v_spec = pl.BlockSpec((D, 1), lambda i: (0, 0))
  r_spec = pl.BlockSpec((1, D), lambda i: (0, 0))

  hs0, als0, ald0 = pl.pallas_call(
      _tc1_body,
      grid=(nb,),
      in_specs=[row_spec, w_spec, w_spec, v_spec, v_spec],
      out_specs=[row_spec, col1_spec, col1_spec],
      out_shape=[
          jax.ShapeDtypeStruct((N, D), jnp.float32),
          jax.ShapeDtypeStruct((N, 1), jnp.float32),
          jax.ShapeDtypeStruct((N, 1), jnp.float32),
      ],
  )(x, w_src0, w_dst0, a_s0c, a_d0c)

  sc_edge = _make_sc_edge_kernel(N, E)

  out2_0, den2_0 = sc_edge(hs0, als0.reshape(N), ald0.reshape(N),
                           src2, dst2)

  h, hs1, als1, ald1 = pl.pallas_call(
      _tc2_body,
      grid=(nb,),
      in_specs=[row_spec, row_spec, col1_spec, col1_spec, row_spec,
                r_spec, w_spec, r_spec, w_spec, v_spec, v_spec],
      out_specs=[row_spec, row_spec, col1_spec, col1_spec],
      out_shape=[
          jax.ShapeDtypeStruct((N, D), jnp.float32),
          jax.ShapeDtypeStruct((N, D), jnp.float32),
          jax.ShapeDtypeStruct((N, 1), jnp.float32),
          jax.ShapeDtypeStruct((N, 1), jnp.float32),
      ],
  )(out2_0[0], out2_0[1], den2_0[0].reshape(N, 1), den2_0[1].reshape(N, 1),
    x, b0r, wl0, bl0r, w1, a_s1c, a_d1c)

  out2_1, den2_1 = sc_edge(hs1, als1.reshape(N), ald1.reshape(N),
                           src2, dst2)

  h2, ssum, sqsum = pl.pallas_call(
      _tc3_body,
      grid=(nb,),
      in_specs=[row_spec, row_spec, col1_spec, col1_spec, row_spec,
                r_spec, w_spec, r_spec],
      out_specs=[row_spec, pl.BlockSpec((nb, D), lambda i: (0, 0)),
                 pl.BlockSpec((nb, D), lambda i: (0, 0))],
      out_shape=[
          jax.ShapeDtypeStruct((N, D), jnp.float32),
          jax.ShapeDtypeStruct((nb, D), jnp.float32),
          jax.ShapeDtypeStruct((nb, D), jnp.float32),
      ],
  )(out2_1[0], out2_1[1], den2_1[0].reshape(N, 1), den2_1[1].reshape(N, 1),
    h, b1r, wl1, bl1r)

  y = pl.pallas_call(
      functools.partial(_tc4_body, float(N * D)),
      grid=(nb,),
      in_specs=[row_spec, pl.BlockSpec((nb, D), lambda i: (0, 0)),
                pl.BlockSpec((nb, D), lambda i: (0, 0)),
                r_spec, r_spec, v_spec, pl.BlockSpec((1, 1), lambda i: (0, 0))],
      out_specs=[col1_spec],
      out_shape=[jax.ShapeDtypeStruct((N, 1), jnp.float32)],
  )(h2, ssum, sqsum, ln_wr, ln_br, wp, bpr)[0]

  return y
